# baseline (speedup 1.0000x reference)
import functools

import jax
import jax.numpy as jnp
from jax import lax
from jax.experimental import pallas as pl
from jax.experimental.pallas import tpu as pltpu
from jax.experimental.pallas import tpu_sc as plsc

H = 128
N_MOL = 1024
APM = 48
N_ATOMS = 1 + N_MOL * APM
N_BONDS = 1 + N_MOL * APM * 4
MAX_NB = 6

A_PAD = 50176
B_PAD = 200704
_CA = 16
_CB = 64
_NS = 98
_NW = 32
_MESH = dict(core_axis_name="c", subcore_axis_name="s")


def _wid():
    return lax.axis_index("s") * 2 + lax.axis_index("c")


def _sc_gather(table, idx, chunk):
    b = idx.shape[0]
    bpw = b // _NW
    nsteps = bpw // chunk
    mesh = plsc.VectorSubcoreMesh(**_MESH)

    @functools.partial(
        pl.kernel, mesh=mesh,
        out_type=jax.ShapeDtypeStruct((b, H), jnp.float32),
        scratch_types=[
            pltpu.VMEM((chunk,), jnp.int32),
            pltpu.VMEM((chunk, H), jnp.float32),
            pltpu.SemaphoreType.DMA,
        ],
    )
    def k(table_hbm, idx_hbm, out_hbm, idx_v, rows_v, sem):
        wid = _wid()
        base = wid * bpw

        def step(g, carry):
            off = base + g * chunk
            pltpu.sync_copy(idx_hbm.at[pl.ds(off, chunk)], idx_v)
            pltpu.async_copy(table_hbm.at[idx_v], rows_v, sem).wait()
            pltpu.sync_copy(rows_v, out_hbm.at[pl.ds(off, chunk)])
            return carry

        lax.fori_loop(0, nsteps, step, 0)

    return k(table, idx)


def _sc_presub(ma, mb, b2a3, b2revb3):
    mesh = plsc.VectorSubcoreMesh(**_MESH)

    @functools.partial(
        pl.kernel, mesh=mesh,
        out_type=jax.ShapeDtypeStruct((B_PAD, H), jnp.float32),
        scratch_types=[
            pltpu.VMEM((_CB,), jnp.int32),
            pltpu.VMEM((_CB,), jnp.int32),
            pltpu.VMEM((_CB, H), jnp.float32),
            pltpu.VMEM((_CB, H), jnp.float32),
            pltpu.VMEM((_CB, H), jnp.float32),
            pltpu.VMEM((_CB, H), jnp.float32),
            pltpu.VMEM((_CB, H), jnp.float32),
            pltpu.SemaphoreType.DMA,
            pltpu.SemaphoreType.DMA,
            pltpu.SemaphoreType.DMA,
            pltpu.SemaphoreType.DMA,
        ],
    )
    def k(ma_hbm, mb_hbm, ia_hbm, ib_hbm, out_hbm, ixa_v, ixb_v,
          ra0, rb0, ra1, rb1, out_v, sa0, sb0, sa1, sb1):
        wid = _wid()
        rbase = wid * (_NS * _CB)

        def compute(ra, rb):
            for i in range(_CB):
                for l in range(8):
                    sl = pl.ds(l * 16, 16)
                    out_v[i, sl] = ra[i, sl] - rb[i, sl]

        def step(g, carry):
            off = rbase + g * _CB
            pltpu.sync_copy(ia_hbm.at[pl.ds(off, _CB)], ixa_v)
            pltpu.sync_copy(ib_hbm.at[pl.ds(off, _CB)], ixb_v)
            pltpu.async_copy(ma_hbm.at[ixa_v], ra0, sa0)
            pltpu.async_copy(mb_hbm.at[ixb_v], rb0, sb0)
            pltpu.make_async_copy(ma_hbm.at[ixa_v], ra0, sa0).wait()
            pltpu.make_async_copy(mb_hbm.at[ixb_v], rb0, sb0).wait()
            compute(ra0, rb0)
            pltpu.sync_copy(out_v, out_hbm.at[pl.ds(off, _CB)])
            return carry

        lax.fori_loop(0, _NS, step, 0)

    return k(ma, mb, b2a3, b2revb3)


def _mm_relu_body(x_ref, wt_ref, o_ref):
    o_ref[...] = jax.nn.relu(
        jnp.dot(x_ref[...], wt_ref[...], preferred_element_type=jnp.float32))


def _mm_relu(x, wt, n_out, bn):
    k = x.shape[1]
    return pl.pallas_call(
        _mm_relu_body,
        grid=(n_out // bn,),
        in_specs=[pl.BlockSpec((bn, k), lambda i: (i, 0)),
                  pl.BlockSpec((k, H), lambda i: (0, 0))],
        out_specs=pl.BlockSpec((bn, H), lambda i: (i, 0)),
        out_shape=jax.ShapeDtypeStruct((n_out, H), jnp.float32),
    )(x, wt)


_BA = 128


def _agg_base_body(nei_ref, base_ref, o_ref):
    x = nei_ref[...].reshape(_BA, 8, H)[:, :MAX_NB, :]
    o_ref[...] = base_ref[...] + x.sum(axis=1) * x.max(axis=1)


def _agg_nb_body(nei_ref, o_ref):
    x = nei_ref[...].reshape(_BA, 8, H)[:, :MAX_NB, :]
    o_ref[...] = x.sum(axis=1) * x.max(axis=1)


def _agg_base(nei8, base):
    return pl.pallas_call(
        _agg_base_body,
        grid=(A_PAD // _BA,),
        in_specs=[pl.BlockSpec((_BA * 8, H), lambda i: (i, 0)),
                  pl.BlockSpec((_BA, H), lambda i: (i, 0))],
        out_specs=pl.BlockSpec((_BA, H), lambda i: (i, 0)),
        out_shape=jax.ShapeDtypeStruct((A_PAD, H), jnp.float32),
    )(nei8, base)


def _agg_nb(nei8):
    return pl.pallas_call(
        _agg_nb_body,
        grid=(A_PAD // _BA,),
        in_specs=[pl.BlockSpec((_BA * 8, H), lambda i: (i, 0))],
        out_specs=pl.BlockSpec((_BA, H), lambda i: (i, 0)),
        out_shape=jax.ShapeDtypeStruct((A_PAD, H), jnp.float32),
    )(nei8)


def _bond_body(pre_ref, ib_ref, wt_ref, o_ref):
    o_ref[...] = jax.nn.relu(
        ib_ref[...] +
        jnp.dot(pre_ref[...], wt_ref[...], preferred_element_type=jnp.float32))


def _bond_update(pre, ib, wh_t):
    bn = 512
    return pl.pallas_call(
        _bond_body,
        grid=(B_PAD // bn,),
        in_specs=[pl.BlockSpec((bn, H), lambda i: (i, 0)),
                  pl.BlockSpec((bn, H), lambda i: (i, 0)),
                  pl.BlockSpec((H, H), lambda i: (0, 0))],
        out_specs=pl.BlockSpec((bn, H), lambda i: (i, 0)),
        out_shape=jax.ShapeDtypeStruct((B_PAD, H), jnp.float32),
    )(pre, ib, wh_t)


_BM5 = 8


def _node_body(agg_ref, ma_ref, ia_ref, l0_ref, l1_ref, l2_ref, gb_ref,
               wif_ref, bif_ref, wir_ref, bir_ref,
               gif_ref, gir_ref, h0_ref):
    node = (jnp.dot(agg_ref[...], l0_ref[...], preferred_element_type=jnp.float32)
            + jnp.dot(ma_ref[...], l1_ref[...], preferred_element_type=jnp.float32)
            + jnp.dot(ia_ref[...], l2_ref[...], preferred_element_type=jnp.float32))
    h0_ref[...] = node.reshape(_BM5, APM, H).max(axis=1)
    msg = jax.nn.relu(node + gb_ref[...])
    gif = jnp.dot(msg, wif_ref[...], preferred_element_type=jnp.float32) + bif_ref[...]
    gir = jnp.dot(msg, wir_ref[...], preferred_element_type=jnp.float32) + bir_ref[...]
    gif_ref[...] = gif.reshape(_BM5, APM, 3 * H).swapaxes(0, 1)
    gir_ref[...] = gir.reshape(_BM5, APM, 3 * H).swapaxes(0, 1)


def _node_project(agg, ma, ia, l0, l1, l2, gbias, wif, bif, wir, bir):
    rows = _BM5 * APM
    wspec = pl.BlockSpec((H, H), lambda i: (0, 0))
    w3spec = pl.BlockSpec((H, 3 * H), lambda i: (0, 0))
    b3spec = pl.BlockSpec((1, 3 * H), lambda i: (0, 0))
    return pl.pallas_call(
        _node_body,
        grid=(N_MOL // _BM5,),
        in_specs=[pl.BlockSpec((rows, H), lambda i: (i, 0)),
                  pl.BlockSpec((rows, H), lambda i: (i, 0)),
                  pl.BlockSpec((rows, H), lambda i: (i, 0)),
                  wspec, wspec, wspec,
                  pl.BlockSpec((1, H), lambda i: (0, 0)),
                  w3spec, b3spec, w3spec, b3spec],
        out_specs=[pl.BlockSpec((APM, _BM5, 3 * H), lambda i: (0, i, 0)),
                   pl.BlockSpec((APM, _BM5, 3 * H), lambda i: (0, i, 0)),
                   pl.BlockSpec((_BM5, H), lambda i: (i, 0))],
        out_shape=[jax.ShapeDtypeStruct((APM, N_MOL, 3 * H), jnp.float32),
                   jax.ShapeDtypeStruct((APM, N_MOL, 3 * H), jnp.float32),
                   jax.ShapeDtypeStruct((N_MOL, H), jnp.float32)],
    )(agg, ma, ia, l0, l1, l2, gbias, wif, bif, wir, bir)


def _gru_body(gif_ref, gir_ref, h0_ref, whf_ref, bhf_ref, whr_ref, bhr_ref,
              of_ref, or_ref, hf_s, hr_s):
    t = pl.program_id(0)

    @pl.when(t == 0)
    def _():
        hf_s[...] = h0_ref[...]
        hr_s[...] = h0_ref[...]

    def step(gi, h, wh_ref, bh_ref):
        gh = jnp.dot(h, wh_ref[...], preferred_element_type=jnp.float32) + bh_ref[...]
        r = jax.nn.sigmoid(gi[:, :H] + gh[:, :H])
        z = jax.nn.sigmoid(gi[:, H:2 * H] + gh[:, H:2 * H])
        n = jnp.tanh(gi[:, 2 * H:] + r * gh[:, 2 * H:])
        return (1.0 - z) * n + z * h

    hf = step(gif_ref[...].reshape(N_MOL, 3 * H), hf_s[...], whf_ref, bhf_ref)
    hr = step(gir_ref[...].reshape(N_MOL, 3 * H), hr_s[...], whr_ref, bhr_ref)
    hf_s[...] = hf
    hr_s[...] = hr
    of_ref[...] = hf.reshape(1, N_MOL, H)
    or_ref[...] = hr.reshape(1, N_MOL, H)


def _gru(gif, gir, h0, whf, bhf, whr, bhr):
    w3spec = pl.BlockSpec((H, 3 * H), lambda t: (0, 0))
    b3spec = pl.BlockSpec((1, 3 * H), lambda t: (0, 0))
    return pl.pallas_call(
        _gru_body,
        grid=(APM,),
        in_specs=[pl.BlockSpec((1, N_MOL, 3 * H), lambda t: (t, 0, 0)),
                  pl.BlockSpec((1, N_MOL, 3 * H), lambda t: (APM - 1 - t, 0, 0)),
                  pl.BlockSpec((N_MOL, H), lambda t: (0, 0)),
                  w3spec, b3spec, w3spec, b3spec],
        out_specs=[pl.BlockSpec((1, N_MOL, H), lambda t: (t, 0, 0)),
                   pl.BlockSpec((1, N_MOL, H), lambda t: (APM - 1 - t, 0, 0))],
        out_shape=[jax.ShapeDtypeStruct((APM, N_MOL, H), jnp.float32),
                   jax.ShapeDtypeStruct((APM, N_MOL, H), jnp.float32)],
        scratch_shapes=[pltpu.VMEM((N_MOL, H), jnp.float32),
                        pltpu.VMEM((N_MOL, H), jnp.float32)],
    )(gif, gir, h0, whf, bhf, whr, bhr)


_BM7 = 128


def _readout_body(of_ref, or_ref, wof_ref, wor_ref, b_ref, o_ref):
    f = of_ref[...].reshape(APM * _BM7, H)
    r = or_ref[...].reshape(APM * _BM7, H)
    ah = jax.nn.relu(
        jnp.dot(f, wof_ref[...], preferred_element_type=jnp.float32)
        + jnp.dot(r, wor_ref[...], preferred_element_type=jnp.float32)
        + b_ref[...])
    o_ref[...] = ah.reshape(APM, _BM7, H).sum(axis=0) * (1.0 / APM)


def _readout(of, orr, wof, wor, b):
    wspec = pl.BlockSpec((H, H), lambda i: (0, 0))
    return pl.pallas_call(
        _readout_body,
        grid=(N_MOL // _BM7,),
        in_specs=[pl.BlockSpec((APM, _BM7, H), lambda i: (0, i, 0)),
                  pl.BlockSpec((APM, _BM7, H), lambda i: (0, i, 0)),
                  wspec, wspec,
                  pl.BlockSpec((1, H), lambda i: (0, 0))],
        out_specs=pl.BlockSpec((_BM7, H), lambda i: (i, 0)),
        out_shape=jax.ShapeDtypeStruct((N_MOL, H), jnp.float32),
    )(of, orr, wof, wor, b)


def kernel(f_atoms, f_bonds, a2b, b2a, b2revb, a_scope, W_i_atom, W_i_bond,
           W_h_0, W_h_1, lr_W, W_o_W, W_o_b, gru_bias, W_ih_f, W_hh_f,
           b_ih_f, b_hh_f, W_ih_r, W_hh_r, b_ih_r, b_hh_r):
    del a_scope

    fa = jnp.pad(f_atoms, ((0, A_PAD - N_ATOMS), (0, 3)))
    fb = jnp.pad(f_bonds, ((0, B_PAD - N_BONDS), (0, 5)))
    wia_t = jnp.pad(W_i_atom.T, ((0, 3), (0, 0)))
    wib_t = jnp.pad(W_i_bond.T, ((0, 5), (0, 0)))
    ia = _mm_relu(fa, wia_t, A_PAD, 512)
    ib = _mm_relu(fb, wib_t, B_PAD, 512)

    a2b8 = jnp.pad(a2b.astype(jnp.int32),
                   ((0, A_PAD - N_ATOMS), (0, 2))).reshape(-1)
    b2a3 = jnp.pad(b2a.astype(jnp.int32), (0, B_PAD - N_BONDS))
    b2revb3 = jnp.pad(b2revb.astype(jnp.int32), (0, B_PAD - N_BONDS))

    ma = ia
    mb = ib
    for wh in (W_h_0, W_h_1):
        ma = _agg_base(_sc_gather(mb, a2b8, 128), ma)
        pre = _sc_presub(ma, mb, b2a3, b2revb3)
        mb = _bond_update(pre, ib, wh.T)

    agg = _agg_nb(_sc_gather(mb, a2b8, 128))

    rows = N_MOL * APM
    agg_s = lax.dynamic_slice(agg, (1, 0), (rows, H))
    ma_s = lax.dynamic_slice(ma, (1, 0), (rows, H))
    ia_s = lax.dynamic_slice(ia, (1, 0), (rows, H))

    l0 = lr_W[:, :H].T
    l1 = lr_W[:, H:2 * H].T
    l2 = lr_W[:, 2 * H:].T
    gif, gir, h0 = _node_project(
        agg_s, ma_s, ia_s, l0, l1, l2, gru_bias.reshape(1, H),
        W_ih_f.T, b_ih_f.reshape(1, 3 * H), W_ih_r.T, b_ih_r.reshape(1, 3 * H))

    of, orr = _gru(gif, gir, h0, W_hh_f.T, b_hh_f.reshape(1, 3 * H),
                   W_hh_r.T, b_hh_r.reshape(1, 3 * H))

    return _readout(of, orr, W_o_W[:, :H].T, W_o_W[:, H:].T,
                    W_o_b.reshape(1, H))


# speedup vs baseline: 3.7929x; 3.7929x over previous
import functools

import jax
import jax.numpy as jnp
from jax import lax
from jax.experimental import pallas as pl
from jax.experimental.pallas import tpu as pltpu
from jax.experimental.pallas import tpu_sc as plsc

H = 128
N_MOL = 1024
APM = 48
N_ATOMS = 1 + N_MOL * APM
N_BONDS = 1 + N_MOL * APM * 4
MAX_NB = 6

A_PAD = 50176
B_PAD = 200704
_CA = 16
_CB = 64
_NS = 98
_NW = 32
_MESH = dict(core_axis_name="c", subcore_axis_name="s")


def _wid():
    return lax.axis_index("s") * 2 + lax.axis_index("c")


def _sc_gather(table, idx, chunk):
    b = idx.shape[0]
    bpw = b // _NW
    nsteps = bpw // chunk
    mesh = plsc.VectorSubcoreMesh(**_MESH)

    @functools.partial(
        pl.kernel, mesh=mesh,
        out_type=jax.ShapeDtypeStruct((b, H), jnp.float32),
        scratch_types=[
            pltpu.VMEM((chunk,), jnp.int32),
            pltpu.VMEM((chunk, H), jnp.float32),
            pltpu.SemaphoreType.DMA,
        ],
    )
    def k(table_hbm, idx_hbm, out_hbm, idx_v, rows_v, sem):
        wid = _wid()
        base = wid * bpw

        def step(g, carry):
            off = base + g * chunk
            pltpu.sync_copy(idx_hbm.at[pl.ds(off, chunk)], idx_v)
            pltpu.async_copy(table_hbm.at[idx_v], rows_v, sem).wait()
            pltpu.sync_copy(rows_v, out_hbm.at[pl.ds(off, chunk)])
            return carry

        lax.fori_loop(0, nsteps, step, 0)

    return k(table, idx)


def _sc_presub(ma, mb, b2a3, b2revb3):
    mesh = plsc.VectorSubcoreMesh(**_MESH)

    @functools.partial(
        pl.kernel, mesh=mesh,
        out_type=jax.ShapeDtypeStruct((B_PAD, H), jnp.float32),
        scratch_types=[
            pltpu.VMEM((_CB,), jnp.int32),
            pltpu.VMEM((_CB,), jnp.int32),
            pltpu.VMEM((_CB, H), jnp.float32),
            pltpu.VMEM((_CB, H), jnp.float32),
            pltpu.VMEM((_CB, H), jnp.float32),
            pltpu.VMEM((_CB, H), jnp.float32),
            pltpu.VMEM((_CB, H), jnp.float32),
            pltpu.SemaphoreType.DMA,
            pltpu.SemaphoreType.DMA,
            pltpu.SemaphoreType.DMA,
            pltpu.SemaphoreType.DMA,
        ],
    )
    def k(ma_hbm, mb_hbm, ia_hbm, ib_hbm, out_hbm, ixa_v, ixb_v,
          ra0, rb0, ra1, rb1, out_v, sa0, sb0, sa1, sb1):
        wid = _wid()
        rbase = wid * (_NS * _CB)

        def compute(ra, rb):
            for i in range(_CB):
                for l in range(8):
                    sl = pl.ds(l * 16, 16)
                    out_v[i, sl] = ra[i, sl] - rb[i, sl]

        def step(g, carry):
            off = rbase + g * _CB
            pltpu.sync_copy(ia_hbm.at[pl.ds(off, _CB)], ixa_v)
            pltpu.sync_copy(ib_hbm.at[pl.ds(off, _CB)], ixb_v)
            pltpu.async_copy(ma_hbm.at[ixa_v], ra0, sa0)
            pltpu.async_copy(mb_hbm.at[ixb_v], rb0, sb0)
            pltpu.make_async_copy(ma_hbm.at[ixa_v], ra0, sa0).wait()
            pltpu.make_async_copy(mb_hbm.at[ixb_v], rb0, sb0).wait()
            compute(ra0, rb0)
            pltpu.sync_copy(out_v, out_hbm.at[pl.ds(off, _CB)])
            return carry

        lax.fori_loop(0, _NS, step, 0)

    return k(ma, mb, b2a3, b2revb3)


def _mm_relu_body(x_ref, wt_ref, o_ref):
    o_ref[...] = jax.nn.relu(
        jnp.dot(x_ref[...], wt_ref[...], preferred_element_type=jnp.float32))


def _mm_relu(x, wt, n_out, bn):
    k = x.shape[1]
    return pl.pallas_call(
        _mm_relu_body,
        grid=(n_out // bn,),
        in_specs=[pl.BlockSpec((bn, k), lambda i: (i, 0)),
                  pl.BlockSpec((k, H), lambda i: (0, 0))],
        out_specs=pl.BlockSpec((bn, H), lambda i: (i, 0)),
        out_shape=jax.ShapeDtypeStruct((n_out, H), jnp.float32),
    )(x, wt)


_BA = 128


def _agg_base_body(nei_ref, base_ref, o_ref):
    x = nei_ref[...].reshape(_BA, 8, H)[:, :MAX_NB, :]
    o_ref[...] = base_ref[...] + x.sum(axis=1) * x.max(axis=1)


def _agg_nb_body(nei_ref, o_ref):
    x = nei_ref[...].reshape(_BA, 8, H)[:, :MAX_NB, :]
    o_ref[...] = x.sum(axis=1) * x.max(axis=1)


def _agg_base(nei8, base):
    return pl.pallas_call(
        _agg_base_body,
        grid=(A_PAD // _BA,),
        in_specs=[pl.BlockSpec((_BA * 8, H), lambda i: (i, 0)),
                  pl.BlockSpec((_BA, H), lambda i: (i, 0))],
        out_specs=pl.BlockSpec((_BA, H), lambda i: (i, 0)),
        out_shape=jax.ShapeDtypeStruct((A_PAD, H), jnp.float32),
    )(nei8, base)


def _agg_nb(nei8):
    return pl.pallas_call(
        _agg_nb_body,
        grid=(A_PAD // _BA,),
        in_specs=[pl.BlockSpec((_BA * 8, H), lambda i: (i, 0))],
        out_specs=pl.BlockSpec((_BA, H), lambda i: (i, 0)),
        out_shape=jax.ShapeDtypeStruct((A_PAD, H), jnp.float32),
    )(nei8)


def _bond_body(pre_ref, ib_ref, wt_ref, o_ref):
    o_ref[...] = jax.nn.relu(
        ib_ref[...] +
        jnp.dot(pre_ref[...], wt_ref[...], preferred_element_type=jnp.float32))


def _bond_update(pre, ib, wh_t):
    bn = 512
    return pl.pallas_call(
        _bond_body,
        grid=(B_PAD // bn,),
        in_specs=[pl.BlockSpec((bn, H), lambda i: (i, 0)),
                  pl.BlockSpec((bn, H), lambda i: (i, 0)),
                  pl.BlockSpec((H, H), lambda i: (0, 0))],
        out_specs=pl.BlockSpec((bn, H), lambda i: (i, 0)),
        out_shape=jax.ShapeDtypeStruct((B_PAD, H), jnp.float32),
    )(pre, ib, wh_t)


_BM5 = 8


def _node_body(agg_ref, ma_ref, ia_ref, l0_ref, l1_ref, l2_ref, gb_ref,
               wif_ref, bif_ref, wir_ref, bir_ref,
               gif_ref, gir_ref, h0_ref):
    node = (jnp.dot(agg_ref[...], l0_ref[...], preferred_element_type=jnp.float32)
            + jnp.dot(ma_ref[...], l1_ref[...], preferred_element_type=jnp.float32)
            + jnp.dot(ia_ref[...], l2_ref[...], preferred_element_type=jnp.float32))
    h0_ref[...] = node.reshape(_BM5, APM, H).max(axis=1)
    msg = jax.nn.relu(node + gb_ref[...])
    gif = jnp.dot(msg, wif_ref[...], preferred_element_type=jnp.float32) + bif_ref[...]
    gir = jnp.dot(msg, wir_ref[...], preferred_element_type=jnp.float32) + bir_ref[...]
    gif_ref[...] = gif.reshape(_BM5, APM, 3 * H).swapaxes(0, 1)
    gir_ref[...] = gir.reshape(_BM5, APM, 3 * H).swapaxes(0, 1)


def _node_project(agg, ma, ia, l0, l1, l2, gbias, wif, bif, wir, bir):
    rows = _BM5 * APM
    wspec = pl.BlockSpec((H, H), lambda i: (0, 0))
    w3spec = pl.BlockSpec((H, 3 * H), lambda i: (0, 0))
    b3spec = pl.BlockSpec((1, 3 * H), lambda i: (0, 0))
    return pl.pallas_call(
        _node_body,
        grid=(N_MOL // _BM5,),
        in_specs=[pl.BlockSpec((rows, H), lambda i: (i, 0)),
                  pl.BlockSpec((rows, H), lambda i: (i, 0)),
                  pl.BlockSpec((rows, H), lambda i: (i, 0)),
                  wspec, wspec, wspec,
                  pl.BlockSpec((1, H), lambda i: (0, 0)),
                  w3spec, b3spec, w3spec, b3spec],
        out_specs=[pl.BlockSpec((APM, _BM5, 3 * H), lambda i: (0, i, 0)),
                   pl.BlockSpec((APM, _BM5, 3 * H), lambda i: (0, i, 0)),
                   pl.BlockSpec((_BM5, H), lambda i: (i, 0))],
        out_shape=[jax.ShapeDtypeStruct((APM, N_MOL, 3 * H), jnp.float32),
                   jax.ShapeDtypeStruct((APM, N_MOL, 3 * H), jnp.float32),
                   jax.ShapeDtypeStruct((N_MOL, H), jnp.float32)],
    )(agg, ma, ia, l0, l1, l2, gbias, wif, bif, wir, bir)


def _gru_body(gif_ref, gir_ref, h0_ref, whf_ref, bhf_ref, whr_ref, bhr_ref,
              of_ref, or_ref, hf_s, hr_s):
    t = pl.program_id(0)

    @pl.when(t == 0)
    def _():
        hf_s[...] = h0_ref[...]
        hr_s[...] = h0_ref[...]

    def step(gi, h, wh_ref, bh_ref):
        gh = jnp.dot(h, wh_ref[...], preferred_element_type=jnp.float32) + bh_ref[...]
        r = jax.nn.sigmoid(gi[:, :H] + gh[:, :H])
        z = jax.nn.sigmoid(gi[:, H:2 * H] + gh[:, H:2 * H])
        n = jnp.tanh(gi[:, 2 * H:] + r * gh[:, 2 * H:])
        return (1.0 - z) * n + z * h

    hf = step(gif_ref[...].reshape(N_MOL, 3 * H), hf_s[...], whf_ref, bhf_ref)
    hr = step(gir_ref[...].reshape(N_MOL, 3 * H), hr_s[...], whr_ref, bhr_ref)
    hf_s[...] = hf
    hr_s[...] = hr
    of_ref[...] = hf.reshape(1, N_MOL, H)
    or_ref[...] = hr.reshape(1, N_MOL, H)


def _gru(gif, gir, h0, whf, bhf, whr, bhr):
    w3spec = pl.BlockSpec((H, 3 * H), lambda t: (0, 0))
    b3spec = pl.BlockSpec((1, 3 * H), lambda t: (0, 0))
    return pl.pallas_call(
        _gru_body,
        grid=(APM,),
        in_specs=[pl.BlockSpec((1, N_MOL, 3 * H), lambda t: (t, 0, 0)),
                  pl.BlockSpec((1, N_MOL, 3 * H), lambda t: (APM - 1 - t, 0, 0)),
                  pl.BlockSpec((N_MOL, H), lambda t: (0, 0)),
                  w3spec, b3spec, w3spec, b3spec],
        out_specs=[pl.BlockSpec((1, N_MOL, H), lambda t: (t, 0, 0)),
                   pl.BlockSpec((1, N_MOL, H), lambda t: (APM - 1 - t, 0, 0))],
        out_shape=[jax.ShapeDtypeStruct((APM, N_MOL, H), jnp.float32),
                   jax.ShapeDtypeStruct((APM, N_MOL, H), jnp.float32)],
        scratch_shapes=[pltpu.VMEM((N_MOL, H), jnp.float32),
                        pltpu.VMEM((N_MOL, H), jnp.float32)],
    )(gif, gir, h0, whf, bhf, whr, bhr)


_BM7 = 128


def _readout_body(of_ref, or_ref, wof_ref, wor_ref, b_ref, o_ref):
    f = of_ref[...].reshape(APM * _BM7, H)
    r = or_ref[...].reshape(APM * _BM7, H)
    ah = jax.nn.relu(
        jnp.dot(f, wof_ref[...], preferred_element_type=jnp.float32)
        + jnp.dot(r, wor_ref[...], preferred_element_type=jnp.float32)
        + b_ref[...])
    o_ref[...] = ah.reshape(APM, _BM7, H).sum(axis=0) * (1.0 / APM)


def _readout(of, orr, wof, wor, b):
    wspec = pl.BlockSpec((H, H), lambda i: (0, 0))
    return pl.pallas_call(
        _readout_body,
        grid=(N_MOL // _BM7,),
        in_specs=[pl.BlockSpec((APM, _BM7, H), lambda i: (0, i, 0)),
                  pl.BlockSpec((APM, _BM7, H), lambda i: (0, i, 0)),
                  wspec, wspec,
                  pl.BlockSpec((1, H), lambda i: (0, 0))],
        out_specs=pl.BlockSpec((_BM7, H), lambda i: (i, 0)),
        out_shape=jax.ShapeDtypeStruct((N_MOL, H), jnp.float32),
    )(of, orr, wof, wor, b)


def kernel(f_atoms, f_bonds, a2b, b2a, b2revb, a_scope, W_i_atom, W_i_bond,
           W_h_0, W_h_1, lr_W, W_o_W, W_o_b, gru_bias, W_ih_f, W_hh_f,
           b_ih_f, b_hh_f, W_ih_r, W_hh_r, b_ih_r, b_hh_r):
    del a_scope

    fa = jnp.pad(f_atoms, ((0, A_PAD - N_ATOMS), (0, 3)))
    fb = jnp.pad(f_bonds, ((0, B_PAD - N_BONDS), (0, 5)))
    wia_t = jnp.pad(W_i_atom.T, ((0, 3), (0, 0)))
    wib_t = jnp.pad(W_i_bond.T, ((0, 5), (0, 0)))
    ia = _mm_relu(fa, wia_t, A_PAD, 512)
    ib = _mm_relu(fb, wib_t, B_PAD, 512)

    arow = jnp.arange(A_PAD, dtype=jnp.int32)[:, None]
    a2b_p = jnp.pad(a2b.astype(jnp.int32), ((0, A_PAD - N_ATOMS), (0, 0)))
    a2b_p = jnp.where(arow < N_ATOMS, a2b_p, arow)
    a2b8 = jnp.concatenate(
        [a2b_p, jnp.broadcast_to(arow, (A_PAD, 2))], axis=1).reshape(-1)
    brow = jnp.arange(B_PAD, dtype=jnp.int32)
    b2a3 = jnp.where(brow < N_BONDS,
                     jnp.pad(b2a.astype(jnp.int32), (0, B_PAD - N_BONDS)),
                     brow % jnp.int32(N_ATOMS))
    b2revb3 = jnp.where(brow < N_BONDS,
                        jnp.pad(b2revb.astype(jnp.int32), (0, B_PAD - N_BONDS)),
                        brow % jnp.int32(N_BONDS))

    ma = ia
    mb = ib
    for wh in (W_h_0, W_h_1):
        ma = _agg_base(_sc_gather(mb, a2b8, 128), ma)
        pre = _sc_presub(ma, mb, b2a3, b2revb3)
        mb = _bond_update(pre, ib, wh.T)

    agg = _agg_nb(_sc_gather(mb, a2b8, 128))

    rows = N_MOL * APM
    agg_s = lax.dynamic_slice(agg, (1, 0), (rows, H))
    ma_s = lax.dynamic_slice(ma, (1, 0), (rows, H))
    ia_s = lax.dynamic_slice(ia, (1, 0), (rows, H))

    l0 = lr_W[:, :H].T
    l1 = lr_W[:, H:2 * H].T
    l2 = lr_W[:, 2 * H:].T
    gif, gir, h0 = _node_project(
        agg_s, ma_s, ia_s, l0, l1, l2, gru_bias.reshape(1, H),
        W_ih_f.T, b_ih_f.reshape(1, 3 * H), W_ih_r.T, b_ih_r.reshape(1, 3 * H))

    of, orr = _gru(gif, gir, h0, W_hh_f.T, b_hh_f.reshape(1, 3 * H),
                   W_hh_r.T, b_hh_r.reshape(1, 3 * H))

    return _readout(of, orr, W_o_W[:, :H].T, W_o_W[:, H:].T,
                    W_o_b.reshape(1, H))


# speedup vs baseline: 4.2322x; 1.1158x over previous
import functools

import jax
import jax.numpy as jnp
from jax import lax
from jax.experimental import pallas as pl
from jax.experimental.pallas import tpu as pltpu
from jax.experimental.pallas import tpu_sc as plsc

H = 128
N_MOL = 1024
APM = 48
N_ATOMS = 1 + N_MOL * APM
N_BONDS = 1 + N_MOL * APM * 4
MAX_NB = 6

A_PAD = 50176
B_PAD = 200704
_CA = 16
_CB = 64
_NS = 98
_NW = 32
_MESH = dict(core_axis_name="c", subcore_axis_name="s")


def _wid():
    return lax.axis_index("s") * 2 + lax.axis_index("c")


def _sc_gather(table, idx, chunk):
    b = idx.shape[0]
    bpw = b // _NW
    nsteps = bpw // chunk
    mesh = plsc.VectorSubcoreMesh(**_MESH)

    @functools.partial(
        pl.kernel, mesh=mesh,
        out_type=jax.ShapeDtypeStruct((b, H), jnp.float32),
        scratch_types=[
            pltpu.VMEM((chunk,), jnp.int32),
            pltpu.VMEM((chunk,), jnp.int32),
            pltpu.VMEM((chunk, H), jnp.float32),
            pltpu.VMEM((chunk, H), jnp.float32),
            pltpu.SemaphoreType.DMA,
            pltpu.SemaphoreType.DMA,
        ],
    )
    def k(table_hbm, idx_hbm, out_hbm, idx0, idx1, rows0, rows1, s0, s1):
        wid = _wid()
        base = wid * bpw

        pltpu.sync_copy(idx_hbm.at[pl.ds(base, chunk)], idx0)
        pltpu.async_copy(table_hbm.at[idx0], rows0, s0)

        def pair(i, carry):
            g = i * 2
            pltpu.sync_copy(idx_hbm.at[pl.ds(base + (g + 1) * chunk, chunk)],
                            idx1)
            pltpu.async_copy(table_hbm.at[idx1], rows1, s1)
            pltpu.make_async_copy(table_hbm.at[idx0], rows0, s0).wait()
            pltpu.sync_copy(rows0, out_hbm.at[pl.ds(base + g * chunk, chunk)])

            @pl.when(g + 2 < nsteps)
            def _():
                pltpu.sync_copy(
                    idx_hbm.at[pl.ds(base + (g + 2) * chunk, chunk)], idx0)
                pltpu.async_copy(table_hbm.at[idx0], rows0, s0)

            pltpu.make_async_copy(table_hbm.at[idx1], rows1, s1).wait()
            pltpu.sync_copy(rows1,
                            out_hbm.at[pl.ds(base + (g + 1) * chunk, chunk)])
            return carry

        lax.fori_loop(0, nsteps // 2, pair, 0)

    return k(table, idx)


def _sc_presub(ma, mb, b2a3, b2revb3):
    mesh = plsc.VectorSubcoreMesh(**_MESH)

    @functools.partial(
        pl.kernel, mesh=mesh,
        out_type=jax.ShapeDtypeStruct((B_PAD, H), jnp.float32),
        scratch_types=[
            pltpu.VMEM((_CB,), jnp.int32),
            pltpu.VMEM((_CB,), jnp.int32),
            pltpu.VMEM((_CB,), jnp.int32),
            pltpu.VMEM((_CB,), jnp.int32),
            pltpu.VMEM((_CB, H), jnp.float32),
            pltpu.VMEM((_CB, H), jnp.float32),
            pltpu.VMEM((_CB, H), jnp.float32),
            pltpu.VMEM((_CB, H), jnp.float32),
            pltpu.VMEM((_CB, H), jnp.float32),
            pltpu.SemaphoreType.DMA,
            pltpu.SemaphoreType.DMA,
            pltpu.SemaphoreType.DMA,
            pltpu.SemaphoreType.DMA,
        ],
    )
    def k(ma_hbm, mb_hbm, ia_hbm, ib_hbm, out_hbm, ixa_v, ixb_v,
          ixa1_v, ixb1_v, ra0, rb0, ra1, rb1, out_v, sa0, sb0, sa1, sb1):
        wid = _wid()
        rbase = wid * (_NS * _CB)

        def compute(ra, rb):
            for i in range(_CB):
                for l in range(8):
                    sl = pl.ds(l * 16, 16)
                    out_v[i, sl] = ra[i, sl] - rb[i, sl]

        def start(g, ixa, ixb, ra, rb, sa, sb):
            off = rbase + g * _CB
            pltpu.sync_copy(ia_hbm.at[pl.ds(off, _CB)], ixa)
            pltpu.sync_copy(ib_hbm.at[pl.ds(off, _CB)], ixb)
            pltpu.async_copy(ma_hbm.at[ixa], ra, sa)
            pltpu.async_copy(mb_hbm.at[ixb], rb, sb)

        def finish(g, ixa, ixb, ra, rb, sa, sb):
            pltpu.make_async_copy(ma_hbm.at[ixa], ra, sa).wait()
            pltpu.make_async_copy(mb_hbm.at[ixb], rb, sb).wait()
            compute(ra, rb)
            pltpu.sync_copy(out_v, out_hbm.at[pl.ds(rbase + g * _CB, _CB)])

        start(0, ixa_v, ixb_v, ra0, rb0, sa0, sb0)

        def pair(i, carry):
            g = i * 2
            start(g + 1, ixa1_v, ixb1_v, ra1, rb1, sa1, sb1)
            finish(g, ixa_v, ixb_v, ra0, rb0, sa0, sb0)

            @pl.when(g + 2 < _NS)
            def _():
                start(g + 2, ixa_v, ixb_v, ra0, rb0, sa0, sb0)

            finish(g + 1, ixa1_v, ixb1_v, ra1, rb1, sa1, sb1)
            return carry

        lax.fori_loop(0, _NS // 2, pair, 0)

    return k(ma, mb, b2a3, b2revb3)


def _mm_relu_body(x_ref, wt_ref, o_ref):
    o_ref[...] = jax.nn.relu(
        jnp.dot(x_ref[...], wt_ref[...], preferred_element_type=jnp.float32))


def _mm_relu(x, wt, n_out, bn):
    k = x.shape[1]
    return pl.pallas_call(
        _mm_relu_body,
        grid=(n_out // bn,),
        in_specs=[pl.BlockSpec((bn, k), lambda i: (i, 0)),
                  pl.BlockSpec((k, H), lambda i: (0, 0))],
        out_specs=pl.BlockSpec((bn, H), lambda i: (i, 0)),
        out_shape=jax.ShapeDtypeStruct((n_out, H), jnp.float32),
    )(x, wt)


_BA = 128


def _agg_base_body(nei_ref, base_ref, o_ref):
    x = nei_ref[...].reshape(_BA, 8, H)[:, :MAX_NB, :]
    o_ref[...] = base_ref[...] + x.sum(axis=1) * x.max(axis=1)


def _agg_nb_body(nei_ref, o_ref):
    x = nei_ref[...].reshape(_BA, 8, H)[:, :MAX_NB, :]
    o_ref[...] = x.sum(axis=1) * x.max(axis=1)


def _agg_base(nei8, base):
    return pl.pallas_call(
        _agg_base_body,
        grid=(A_PAD // _BA,),
        in_specs=[pl.BlockSpec((_BA * 8, H), lambda i: (i, 0)),
                  pl.BlockSpec((_BA, H), lambda i: (i, 0))],
        out_specs=pl.BlockSpec((_BA, H), lambda i: (i, 0)),
        out_shape=jax.ShapeDtypeStruct((A_PAD, H), jnp.float32),
    )(nei8, base)


def _agg_nb(nei8):
    return pl.pallas_call(
        _agg_nb_body,
        grid=(A_PAD // _BA,),
        in_specs=[pl.BlockSpec((_BA * 8, H), lambda i: (i, 0))],
        out_specs=pl.BlockSpec((_BA, H), lambda i: (i, 0)),
        out_shape=jax.ShapeDtypeStruct((A_PAD, H), jnp.float32),
    )(nei8)


def _bond_body(pre_ref, ib_ref, wt_ref, o_ref):
    o_ref[...] = jax.nn.relu(
        ib_ref[...] +
        jnp.dot(pre_ref[...], wt_ref[...], preferred_element_type=jnp.float32))


def _bond_update(pre, ib, wh_t):
    bn = 512
    return pl.pallas_call(
        _bond_body,
        grid=(B_PAD // bn,),
        in_specs=[pl.BlockSpec((bn, H), lambda i: (i, 0)),
                  pl.BlockSpec((bn, H), lambda i: (i, 0)),
                  pl.BlockSpec((H, H), lambda i: (0, 0))],
        out_specs=pl.BlockSpec((bn, H), lambda i: (i, 0)),
        out_shape=jax.ShapeDtypeStruct((B_PAD, H), jnp.float32),
    )(pre, ib, wh_t)


_BM5 = 8


def _node_body(agg_ref, ma_ref, ia_ref, l0_ref, l1_ref, l2_ref, gb_ref,
               wif_ref, bif_ref, wir_ref, bir_ref,
               gif_ref, gir_ref, h0_ref):
    node = (jnp.dot(agg_ref[...], l0_ref[...], preferred_element_type=jnp.float32)
            + jnp.dot(ma_ref[...], l1_ref[...], preferred_element_type=jnp.float32)
            + jnp.dot(ia_ref[...], l2_ref[...], preferred_element_type=jnp.float32))
    h0_ref[...] = node.reshape(_BM5, APM, H).max(axis=1)
    msg = jax.nn.relu(node + gb_ref[...])
    gif = jnp.dot(msg, wif_ref[...], preferred_element_type=jnp.float32) + bif_ref[...]
    gir = jnp.dot(msg, wir_ref[...], preferred_element_type=jnp.float32) + bir_ref[...]
    gif_ref[...] = gif.reshape(_BM5, APM, 3 * H).swapaxes(0, 1)
    gir_ref[...] = gir.reshape(_BM5, APM, 3 * H).swapaxes(0, 1)


def _node_project(agg, ma, ia, l0, l1, l2, gbias, wif, bif, wir, bir):
    rows = _BM5 * APM
    wspec = pl.BlockSpec((H, H), lambda i: (0, 0))
    w3spec = pl.BlockSpec((H, 3 * H), lambda i: (0, 0))
    b3spec = pl.BlockSpec((1, 3 * H), lambda i: (0, 0))
    return pl.pallas_call(
        _node_body,
        grid=(N_MOL // _BM5,),
        in_specs=[pl.BlockSpec((rows, H), lambda i: (i, 0)),
                  pl.BlockSpec((rows, H), lambda i: (i, 0)),
                  pl.BlockSpec((rows, H), lambda i: (i, 0)),
                  wspec, wspec, wspec,
                  pl.BlockSpec((1, H), lambda i: (0, 0)),
                  w3spec, b3spec, w3spec, b3spec],
        out_specs=[pl.BlockSpec((APM, _BM5, 3 * H), lambda i: (0, i, 0)),
                   pl.BlockSpec((APM, _BM5, 3 * H), lambda i: (0, i, 0)),
                   pl.BlockSpec((_BM5, H), lambda i: (i, 0))],
        out_shape=[jax.ShapeDtypeStruct((APM, N_MOL, 3 * H), jnp.float32),
                   jax.ShapeDtypeStruct((APM, N_MOL, 3 * H), jnp.float32),
                   jax.ShapeDtypeStruct((N_MOL, H), jnp.float32)],
    )(agg, ma, ia, l0, l1, l2, gbias, wif, bif, wir, bir)


def _gru_body(gif_ref, gir_ref, h0_ref, whf_ref, bhf_ref, whr_ref, bhr_ref,
              of_ref, or_ref, hf_s, hr_s):
    t = pl.program_id(0)

    @pl.when(t == 0)
    def _():
        hf_s[...] = h0_ref[...]
        hr_s[...] = h0_ref[...]

    def step(gi, h, wh_ref, bh_ref):
        gh = jnp.dot(h, wh_ref[...], preferred_element_type=jnp.float32) + bh_ref[...]
        r = jax.nn.sigmoid(gi[:, :H] + gh[:, :H])
        z = jax.nn.sigmoid(gi[:, H:2 * H] + gh[:, H:2 * H])
        n = jnp.tanh(gi[:, 2 * H:] + r * gh[:, 2 * H:])
        return (1.0 - z) * n + z * h

    hf = step(gif_ref[...].reshape(N_MOL, 3 * H), hf_s[...], whf_ref, bhf_ref)
    hr = step(gir_ref[...].reshape(N_MOL, 3 * H), hr_s[...], whr_ref, bhr_ref)
    hf_s[...] = hf
    hr_s[...] = hr
    of_ref[...] = hf.reshape(1, N_MOL, H)
    or_ref[...] = hr.reshape(1, N_MOL, H)


def _gru(gif, gir, h0, whf, bhf, whr, bhr):
    w3spec = pl.BlockSpec((H, 3 * H), lambda t: (0, 0))
    b3spec = pl.BlockSpec((1, 3 * H), lambda t: (0, 0))
    return pl.pallas_call(
        _gru_body,
        grid=(APM,),
        in_specs=[pl.BlockSpec((1, N_MOL, 3 * H), lambda t: (t, 0, 0)),
                  pl.BlockSpec((1, N_MOL, 3 * H), lambda t: (APM - 1 - t, 0, 0)),
                  pl.BlockSpec((N_MOL, H), lambda t: (0, 0)),
                  w3spec, b3spec, w3spec, b3spec],
        out_specs=[pl.BlockSpec((1, N_MOL, H), lambda t: (t, 0, 0)),
                   pl.BlockSpec((1, N_MOL, H), lambda t: (APM - 1 - t, 0, 0))],
        out_shape=[jax.ShapeDtypeStruct((APM, N_MOL, H), jnp.float32),
                   jax.ShapeDtypeStruct((APM, N_MOL, H), jnp.float32)],
        scratch_shapes=[pltpu.VMEM((N_MOL, H), jnp.float32),
                        pltpu.VMEM((N_MOL, H), jnp.float32)],
    )(gif, gir, h0, whf, bhf, whr, bhr)


_BM7 = 128


def _readout_body(of_ref, or_ref, wof_ref, wor_ref, b_ref, o_ref):
    f = of_ref[...].reshape(APM * _BM7, H)
    r = or_ref[...].reshape(APM * _BM7, H)
    ah = jax.nn.relu(
        jnp.dot(f, wof_ref[...], preferred_element_type=jnp.float32)
        + jnp.dot(r, wor_ref[...], preferred_element_type=jnp.float32)
        + b_ref[...])
    o_ref[...] = ah.reshape(APM, _BM7, H).sum(axis=0) * (1.0 / APM)


def _readout(of, orr, wof, wor, b):
    wspec = pl.BlockSpec((H, H), lambda i: (0, 0))
    return pl.pallas_call(
        _readout_body,
        grid=(N_MOL // _BM7,),
        in_specs=[pl.BlockSpec((APM, _BM7, H), lambda i: (0, i, 0)),
                  pl.BlockSpec((APM, _BM7, H), lambda i: (0, i, 0)),
                  wspec, wspec,
                  pl.BlockSpec((1, H), lambda i: (0, 0))],
        out_specs=pl.BlockSpec((_BM7, H), lambda i: (i, 0)),
        out_shape=jax.ShapeDtypeStruct((N_MOL, H), jnp.float32),
    )(of, orr, wof, wor, b)


def kernel(f_atoms, f_bonds, a2b, b2a, b2revb, a_scope, W_i_atom, W_i_bond,
           W_h_0, W_h_1, lr_W, W_o_W, W_o_b, gru_bias, W_ih_f, W_hh_f,
           b_ih_f, b_hh_f, W_ih_r, W_hh_r, b_ih_r, b_hh_r):
    del a_scope

    fa = jnp.pad(f_atoms, ((0, A_PAD - N_ATOMS), (0, 3)))
    fb = jnp.pad(f_bonds, ((0, B_PAD - N_BONDS), (0, 5)))
    wia_t = jnp.pad(W_i_atom.T, ((0, 3), (0, 0)))
    wib_t = jnp.pad(W_i_bond.T, ((0, 5), (0, 0)))
    ia = _mm_relu(fa, wia_t, A_PAD, 512)
    ib = _mm_relu(fb, wib_t, B_PAD, 512)

    arow = jnp.arange(A_PAD, dtype=jnp.int32)[:, None]
    a2b_p = jnp.pad(a2b.astype(jnp.int32), ((0, A_PAD - N_ATOMS), (0, 0)))
    a2b_p = jnp.where(arow < N_ATOMS, a2b_p, arow)
    a2b8 = jnp.concatenate(
        [a2b_p, jnp.broadcast_to(arow, (A_PAD, 2))], axis=1).reshape(-1)
    brow = jnp.arange(B_PAD, dtype=jnp.int32)
    b2a3 = jnp.where(brow < N_BONDS,
                     jnp.pad(b2a.astype(jnp.int32), (0, B_PAD - N_BONDS)),
                     brow % jnp.int32(N_ATOMS))
    b2revb3 = jnp.where(brow < N_BONDS,
                        jnp.pad(b2revb.astype(jnp.int32), (0, B_PAD - N_BONDS)),
                        brow % jnp.int32(N_BONDS))

    ma = ia
    mb = ib
    for wh in (W_h_0, W_h_1):
        ma = _agg_base(_sc_gather(mb, a2b8, 128), ma)
        pre = _sc_presub(ma, mb, b2a3, b2revb3)
        mb = _bond_update(pre, ib, wh.T)

    agg = _agg_nb(_sc_gather(mb, a2b8, 128))

    rows = N_MOL * APM
    agg_s = lax.dynamic_slice(agg, (1, 0), (rows, H))
    ma_s = lax.dynamic_slice(ma, (1, 0), (rows, H))
    ia_s = lax.dynamic_slice(ia, (1, 0), (rows, H))

    l0 = lr_W[:, :H].T
    l1 = lr_W[:, H:2 * H].T
    l2 = lr_W[:, 2 * H:].T
    gif, gir, h0 = _node_project(
        agg_s, ma_s, ia_s, l0, l1, l2, gru_bias.reshape(1, H),
        W_ih_f.T, b_ih_f.reshape(1, 3 * H), W_ih_r.T, b_ih_r.reshape(1, 3 * H))

    of, orr = _gru(gif, gir, h0, W_hh_f.T, b_hh_f.reshape(1, 3 * H),
                   W_hh_r.T, b_hh_r.reshape(1, 3 * H))

    return _readout(of, orr, W_o_W[:, :H].T, W_o_W[:, H:].T,
                    W_o_b.reshape(1, H))


# speedup vs baseline: 5.3049x; 1.2535x over previous
import functools

import jax
import jax.numpy as jnp
from jax import lax
from jax.experimental import pallas as pl
from jax.experimental.pallas import tpu as pltpu
from jax.experimental.pallas import tpu_sc as plsc

H = 128
N_MOL = 1024
APM = 48
N_ATOMS = 1 + N_MOL * APM
N_BONDS = 1 + N_MOL * APM * 4
MAX_NB = 6

A_PAD = 50176
B_PAD = 200704
_CA = 16
_CB = 64
_NS = 98
_NW = 32
_MESH = dict(core_axis_name="c", subcore_axis_name="s")


def _wid():
    return lax.axis_index("s") * 2 + lax.axis_index("c")


def _sc_gather(table, idx, chunk):
    b = idx.shape[0]
    bpw = b // _NW
    nsteps = bpw // chunk
    mesh = plsc.VectorSubcoreMesh(**_MESH)

    @functools.partial(
        pl.kernel, mesh=mesh,
        out_type=jax.ShapeDtypeStruct((b, H), jnp.float32),
        scratch_types=[
            pltpu.VMEM((chunk,), jnp.int32),
            pltpu.VMEM((chunk,), jnp.int32),
            pltpu.VMEM((chunk, H), jnp.float32),
            pltpu.VMEM((chunk, H), jnp.float32),
            pltpu.SemaphoreType.DMA,
            pltpu.SemaphoreType.DMA,
        ],
    )
    def k(table_hbm, idx_hbm, out_hbm, idx0, idx1, rows0, rows1, s0, s1):
        wid = _wid()
        base = wid * bpw

        pltpu.sync_copy(idx_hbm.at[pl.ds(base, chunk)], idx0)
        pltpu.async_copy(table_hbm.at[idx0], rows0, s0)

        def pair(i, carry):
            g = i * 2
            pltpu.sync_copy(idx_hbm.at[pl.ds(base + (g + 1) * chunk, chunk)],
                            idx1)
            pltpu.async_copy(table_hbm.at[idx1], rows1, s1)
            pltpu.make_async_copy(table_hbm.at[idx0], rows0, s0).wait()
            pltpu.sync_copy(rows0, out_hbm.at[pl.ds(base + g * chunk, chunk)])

            @pl.when(g + 2 < nsteps)
            def _():
                pltpu.sync_copy(
                    idx_hbm.at[pl.ds(base + (g + 2) * chunk, chunk)], idx0)
                pltpu.async_copy(table_hbm.at[idx0], rows0, s0)

            pltpu.make_async_copy(table_hbm.at[idx1], rows1, s1).wait()
            pltpu.sync_copy(rows1,
                            out_hbm.at[pl.ds(base + (g + 1) * chunk, chunk)])
            return carry

        lax.fori_loop(0, nsteps // 2, pair, 0)

    return k(table, idx)


def _sc_presub(ma, mb, b2a3, b2revb3):
    mesh = plsc.VectorSubcoreMesh(**_MESH)

    @functools.partial(
        pl.kernel, mesh=mesh,
        out_type=jax.ShapeDtypeStruct((B_PAD, H), jnp.float32),
        scratch_types=[
            pltpu.VMEM((_CB,), jnp.int32),
            pltpu.VMEM((_CB,), jnp.int32),
            pltpu.VMEM((_CB,), jnp.int32),
            pltpu.VMEM((_CB,), jnp.int32),
            pltpu.VMEM((_CB, H), jnp.float32),
            pltpu.VMEM((_CB, H), jnp.float32),
            pltpu.VMEM((_CB, H), jnp.float32),
            pltpu.VMEM((_CB, H), jnp.float32),
            pltpu.VMEM((_CB, H), jnp.float32),
            pltpu.SemaphoreType.DMA,
            pltpu.SemaphoreType.DMA,
            pltpu.SemaphoreType.DMA,
            pltpu.SemaphoreType.DMA,
        ],
    )
    def k(ma_hbm, mb_hbm, ia_hbm, ib_hbm, out_hbm, ixa_v, ixb_v,
          ixa1_v, ixb1_v, ra0, rb0, ra1, rb1, out_v, sa0, sb0, sa1, sb1):
        wid = _wid()
        rbase = wid * (_NS * _CB)

        def compute(ra, rb):
            for i in range(_CB):
                for l in range(8):
                    sl = pl.ds(l * 16, 16)
                    out_v[i, sl] = ra[i, sl] - rb[i, sl]

        def start(g, ixa, ixb, ra, rb, sa, sb):
            off = rbase + g * _CB
            pltpu.sync_copy(ia_hbm.at[pl.ds(off, _CB)], ixa)
            pltpu.sync_copy(ib_hbm.at[pl.ds(off, _CB)], ixb)
            pltpu.async_copy(ma_hbm.at[ixa], ra, sa)
            pltpu.async_copy(mb_hbm.at[ixb], rb, sb)

        def finish(g, ixa, ixb, ra, rb, sa, sb):
            pltpu.make_async_copy(ma_hbm.at[ixa], ra, sa).wait()
            pltpu.make_async_copy(mb_hbm.at[ixb], rb, sb).wait()
            compute(ra, rb)
            pltpu.sync_copy(out_v, out_hbm.at[pl.ds(rbase + g * _CB, _CB)])

        start(0, ixa_v, ixb_v, ra0, rb0, sa0, sb0)

        def pair(i, carry):
            g = i * 2
            start(g + 1, ixa1_v, ixb1_v, ra1, rb1, sa1, sb1)
            finish(g, ixa_v, ixb_v, ra0, rb0, sa0, sb0)

            @pl.when(g + 2 < _NS)
            def _():
                start(g + 2, ixa_v, ixb_v, ra0, rb0, sa0, sb0)

            finish(g + 1, ixa1_v, ixb1_v, ra1, rb1, sa1, sb1)
            return carry

        lax.fori_loop(0, _NS // 2, pair, 0)

    return k(ma, mb, b2a3, b2revb3)


def _mm_relu_body(x_ref, wt_ref, o_ref):
    o_ref[...] = jax.nn.relu(
        lax.dot_general(x_ref[...], wt_ref[...],
                        (((0,), (0,)), ((), ())),
                        preferred_element_type=jnp.float32))


def _mm_relu(xT, wt, n_out, bn):
    k = xT.shape[0]
    return pl.pallas_call(
        _mm_relu_body,
        grid=(n_out // bn,),
        in_specs=[pl.BlockSpec((k, bn), lambda i: (0, i)),
                  pl.BlockSpec((k, H), lambda i: (0, 0))],
        out_specs=pl.BlockSpec((bn, H), lambda i: (i, 0)),
        out_shape=jax.ShapeDtypeStruct((n_out, H), jnp.float32),
    )(xT, wt)


_BA = 128


def _agg_base_body(nei_ref, base_ref, o_ref):
    x = nei_ref[...].reshape(_BA, 8, H)[:, :MAX_NB, :]
    o_ref[...] = base_ref[...] + x.sum(axis=1) * x.max(axis=1)


def _agg_nb_body(nei_ref, o_ref):
    x = nei_ref[...].reshape(_BA, 8, H)[:, :MAX_NB, :]
    o_ref[...] = x.sum(axis=1) * x.max(axis=1)


def _agg_base(nei8, base):
    return pl.pallas_call(
        _agg_base_body,
        grid=(A_PAD // _BA,),
        in_specs=[pl.BlockSpec((_BA * 8, H), lambda i: (i, 0)),
                  pl.BlockSpec((_BA, H), lambda i: (i, 0))],
        out_specs=pl.BlockSpec((_BA, H), lambda i: (i, 0)),
        out_shape=jax.ShapeDtypeStruct((A_PAD, H), jnp.float32),
    )(nei8, base)


def _agg_nb(nei8):
    return pl.pallas_call(
        _agg_nb_body,
        grid=(A_PAD // _BA,),
        in_specs=[pl.BlockSpec((_BA * 8, H), lambda i: (i, 0))],
        out_specs=pl.BlockSpec((_BA, H), lambda i: (i, 0)),
        out_shape=jax.ShapeDtypeStruct((A_PAD, H), jnp.float32),
    )(nei8)


def _bond_body(pre_ref, ib_ref, wt_ref, o_ref):
    o_ref[...] = jax.nn.relu(
        ib_ref[...] +
        jnp.dot(pre_ref[...], wt_ref[...], preferred_element_type=jnp.float32))


def _bond_update(pre, ib, wh_t):
    bn = 512
    return pl.pallas_call(
        _bond_body,
        grid=(B_PAD // bn,),
        in_specs=[pl.BlockSpec((bn, H), lambda i: (i, 0)),
                  pl.BlockSpec((bn, H), lambda i: (i, 0)),
                  pl.BlockSpec((H, H), lambda i: (0, 0))],
        out_specs=pl.BlockSpec((bn, H), lambda i: (i, 0)),
        out_shape=jax.ShapeDtypeStruct((B_PAD, H), jnp.float32),
    )(pre, ib, wh_t)


_BM5 = 8


def _node_body(agg_ref, ma_ref, ia_ref, l0_ref, l1_ref, l2_ref, gb_ref,
               wif_ref, bif_ref, wir_ref, bir_ref,
               gif_ref, gir_ref, h0_ref):
    node = (jnp.dot(agg_ref[...], l0_ref[...], preferred_element_type=jnp.float32)
            + jnp.dot(ma_ref[...], l1_ref[...], preferred_element_type=jnp.float32)
            + jnp.dot(ia_ref[...], l2_ref[...], preferred_element_type=jnp.float32))
    h0_ref[...] = node.reshape(_BM5, APM, H).max(axis=1)
    msg = jax.nn.relu(node + gb_ref[...])
    gif = jnp.dot(msg, wif_ref[...], preferred_element_type=jnp.float32) + bif_ref[...]
    gir = jnp.dot(msg, wir_ref[...], preferred_element_type=jnp.float32) + bir_ref[...]
    gif_ref[...] = gif.reshape(_BM5, APM, 3 * H).swapaxes(0, 1)
    gir_ref[...] = gir.reshape(_BM5, APM, 3 * H).swapaxes(0, 1)


def _node_project(agg, ma, ia, l0, l1, l2, gbias, wif, bif, wir, bir):
    rows = _BM5 * APM
    wspec = pl.BlockSpec((H, H), lambda i: (0, 0))
    w3spec = pl.BlockSpec((H, 3 * H), lambda i: (0, 0))
    b3spec = pl.BlockSpec((1, 3 * H), lambda i: (0, 0))
    return pl.pallas_call(
        _node_body,
        grid=(N_MOL // _BM5,),
        in_specs=[pl.BlockSpec((rows, H), lambda i: (i, 0)),
                  pl.BlockSpec((rows, H), lambda i: (i, 0)),
                  pl.BlockSpec((rows, H), lambda i: (i, 0)),
                  wspec, wspec, wspec,
                  pl.BlockSpec((1, H), lambda i: (0, 0)),
                  w3spec, b3spec, w3spec, b3spec],
        out_specs=[pl.BlockSpec((APM, _BM5, 3 * H), lambda i: (0, i, 0)),
                   pl.BlockSpec((APM, _BM5, 3 * H), lambda i: (0, i, 0)),
                   pl.BlockSpec((_BM5, H), lambda i: (i, 0))],
        out_shape=[jax.ShapeDtypeStruct((APM, N_MOL, 3 * H), jnp.float32),
                   jax.ShapeDtypeStruct((APM, N_MOL, 3 * H), jnp.float32),
                   jax.ShapeDtypeStruct((N_MOL, H), jnp.float32)],
    )(agg, ma, ia, l0, l1, l2, gbias, wif, bif, wir, bir)


def _gru_body(gif_ref, gir_ref, h0_ref, whf_ref, bhf_ref, whr_ref, bhr_ref,
              of_ref, or_ref, hf_s, hr_s):
    t = pl.program_id(0)

    @pl.when(t == 0)
    def _():
        hf_s[...] = h0_ref[...]
        hr_s[...] = h0_ref[...]

    def step(gi, h, wh_ref, bh_ref):
        gh = jnp.dot(h, wh_ref[...], preferred_element_type=jnp.float32) + bh_ref[...]
        r = jax.nn.sigmoid(gi[:, :H] + gh[:, :H])
        z = jax.nn.sigmoid(gi[:, H:2 * H] + gh[:, H:2 * H])
        n = jnp.tanh(gi[:, 2 * H:] + r * gh[:, 2 * H:])
        return (1.0 - z) * n + z * h

    hf = step(gif_ref[...].reshape(N_MOL, 3 * H), hf_s[...], whf_ref, bhf_ref)
    hr = step(gir_ref[...].reshape(N_MOL, 3 * H), hr_s[...], whr_ref, bhr_ref)
    hf_s[...] = hf
    hr_s[...] = hr
    of_ref[...] = hf.reshape(1, N_MOL, H)
    or_ref[...] = hr.reshape(1, N_MOL, H)


def _gru(gif, gir, h0, whf, bhf, whr, bhr):
    w3spec = pl.BlockSpec((H, 3 * H), lambda t: (0, 0))
    b3spec = pl.BlockSpec((1, 3 * H), lambda t: (0, 0))
    return pl.pallas_call(
        _gru_body,
        grid=(APM,),
        in_specs=[pl.BlockSpec((1, N_MOL, 3 * H), lambda t: (t, 0, 0)),
                  pl.BlockSpec((1, N_MOL, 3 * H), lambda t: (APM - 1 - t, 0, 0)),
                  pl.BlockSpec((N_MOL, H), lambda t: (0, 0)),
                  w3spec, b3spec, w3spec, b3spec],
        out_specs=[pl.BlockSpec((1, N_MOL, H), lambda t: (t, 0, 0)),
                   pl.BlockSpec((1, N_MOL, H), lambda t: (APM - 1 - t, 0, 0))],
        out_shape=[jax.ShapeDtypeStruct((APM, N_MOL, H), jnp.float32),
                   jax.ShapeDtypeStruct((APM, N_MOL, H), jnp.float32)],
        scratch_shapes=[pltpu.VMEM((N_MOL, H), jnp.float32),
                        pltpu.VMEM((N_MOL, H), jnp.float32)],
    )(gif, gir, h0, whf, bhf, whr, bhr)


_BM7 = 128


def _readout_body(of_ref, or_ref, wof_ref, wor_ref, b_ref, o_ref):
    f = of_ref[...].reshape(APM * _BM7, H)
    r = or_ref[...].reshape(APM * _BM7, H)
    ah = jax.nn.relu(
        jnp.dot(f, wof_ref[...], preferred_element_type=jnp.float32)
        + jnp.dot(r, wor_ref[...], preferred_element_type=jnp.float32)
        + b_ref[...])
    o_ref[...] = ah.reshape(APM, _BM7, H).sum(axis=0) * (1.0 / APM)


def _readout(of, orr, wof, wor, b):
    wspec = pl.BlockSpec((H, H), lambda i: (0, 0))
    return pl.pallas_call(
        _readout_body,
        grid=(N_MOL // _BM7,),
        in_specs=[pl.BlockSpec((APM, _BM7, H), lambda i: (0, i, 0)),
                  pl.BlockSpec((APM, _BM7, H), lambda i: (0, i, 0)),
                  wspec, wspec,
                  pl.BlockSpec((1, H), lambda i: (0, 0))],
        out_specs=pl.BlockSpec((_BM7, H), lambda i: (i, 0)),
        out_shape=jax.ShapeDtypeStruct((N_MOL, H), jnp.float32),
    )(of, orr, wof, wor, b)


def kernel(f_atoms, f_bonds, a2b, b2a, b2revb, a_scope, W_i_atom, W_i_bond,
           W_h_0, W_h_1, lr_W, W_o_W, W_o_b, gru_bias, W_ih_f, W_hh_f,
           b_ih_f, b_hh_f, W_ih_r, W_hh_r, b_ih_r, b_hh_r):
    del a_scope

    faT = jnp.pad(f_atoms.T, ((0, 3), (0, A_PAD - N_ATOMS)))
    fbT = jnp.pad(f_bonds.T, ((0, 5), (0, B_PAD - N_BONDS)))
    wia_t = jnp.pad(W_i_atom.T, ((0, 3), (0, 0)))
    wib_t = jnp.pad(W_i_bond.T, ((0, 5), (0, 0)))
    ia = _mm_relu(faT, wia_t, A_PAD, 512)
    ib = _mm_relu(fbT, wib_t, B_PAD, 512)

    arow = jnp.arange(A_PAD, dtype=jnp.int32)[:, None]
    a2b_p = jnp.pad(a2b.astype(jnp.int32), ((0, A_PAD - N_ATOMS), (0, 0)))
    a2b_p = jnp.where(arow < N_ATOMS, a2b_p, arow)
    a2b8 = jnp.concatenate(
        [a2b_p, jnp.broadcast_to(arow, (A_PAD, 2))], axis=1).reshape(-1)
    brow = jnp.arange(B_PAD, dtype=jnp.int32)
    b2a3 = jnp.where(brow < N_BONDS,
                     jnp.pad(b2a.astype(jnp.int32), (0, B_PAD - N_BONDS)),
                     brow % jnp.int32(N_ATOMS))
    b2revb3 = jnp.where(brow < N_BONDS,
                        jnp.pad(b2revb.astype(jnp.int32), (0, B_PAD - N_BONDS)),
                        brow % jnp.int32(N_BONDS))

    ma = ia
    mb = ib
    for wh in (W_h_0, W_h_1):
        ma = _agg_base(_sc_gather(mb, a2b8, 128), ma)
        pre = _sc_presub(ma, mb, b2a3, b2revb3)
        mb = _bond_update(pre, ib, wh.T)

    agg = _agg_nb(_sc_gather(mb, a2b8, 128))

    rows = N_MOL * APM
    agg_s = lax.dynamic_slice(agg, (1, 0), (rows, H))
    ma_s = lax.dynamic_slice(ma, (1, 0), (rows, H))
    ia_s = lax.dynamic_slice(ia, (1, 0), (rows, H))

    l0 = lr_W[:, :H].T
    l1 = lr_W[:, H:2 * H].T
    l2 = lr_W[:, 2 * H:].T
    gif, gir, h0 = _node_project(
        agg_s, ma_s, ia_s, l0, l1, l2, gru_bias.reshape(1, H),
        W_ih_f.T, b_ih_f.reshape(1, 3 * H), W_ih_r.T, b_ih_r.reshape(1, 3 * H))

    of, orr = _gru(gif, gir, h0, W_hh_f.T, b_hh_f.reshape(1, 3 * H),
                   W_hh_r.T, b_hh_r.reshape(1, 3 * H))

    return _readout(of, orr, W_o_W[:, :H].T, W_o_W[:, H:].T,
                    W_o_b.reshape(1, H))


# speedup vs baseline: 5.4069x; 1.0192x over previous
import functools

import jax
import jax.numpy as jnp
from jax import lax
from jax.experimental import pallas as pl
from jax.experimental.pallas import tpu as pltpu
from jax.experimental.pallas import tpu_sc as plsc

H = 128
N_MOL = 1024
APM = 48
N_ATOMS = 1 + N_MOL * APM
N_BONDS = 1 + N_MOL * APM * 4
MAX_NB = 6

A_PAD = 50176
B_PAD = 200704
_CA = 16
_CB = 64
_NS = 98
_NW = 32
_MESH = dict(core_axis_name="c", subcore_axis_name="s")


def _wid():
    return lax.axis_index("s") * 2 + lax.axis_index("c")


def _sc_gather(table, idx, chunk):
    b = idx.shape[0]
    bpw = b // _NW
    nsteps = bpw // chunk
    mesh = plsc.VectorSubcoreMesh(**_MESH)

    @functools.partial(
        pl.kernel, mesh=mesh,
        out_type=jax.ShapeDtypeStruct((b, H), jnp.float32),
        scratch_types=[
            pltpu.VMEM((chunk,), jnp.int32),
            pltpu.VMEM((chunk,), jnp.int32),
            pltpu.VMEM((chunk, H), jnp.float32),
            pltpu.VMEM((chunk, H), jnp.float32),
            pltpu.SemaphoreType.DMA,
            pltpu.SemaphoreType.DMA,
        ],
    )
    def k(table_hbm, idx_hbm, out_hbm, idx0, idx1, rows0, rows1, s0, s1):
        wid = _wid()
        base = wid * bpw

        pltpu.sync_copy(idx_hbm.at[pl.ds(base, chunk)], idx0)
        pltpu.async_copy(table_hbm.at[idx0], rows0, s0)

        def pair(i, carry):
            g = i * 2
            pltpu.sync_copy(idx_hbm.at[pl.ds(base + (g + 1) * chunk, chunk)],
                            idx1)
            pltpu.async_copy(table_hbm.at[idx1], rows1, s1)
            pltpu.make_async_copy(table_hbm.at[idx0], rows0, s0).wait()
            pltpu.sync_copy(rows0, out_hbm.at[pl.ds(base + g * chunk, chunk)])

            @pl.when(g + 2 < nsteps)
            def _():
                pltpu.sync_copy(
                    idx_hbm.at[pl.ds(base + (g + 2) * chunk, chunk)], idx0)
                pltpu.async_copy(table_hbm.at[idx0], rows0, s0)

            pltpu.make_async_copy(table_hbm.at[idx1], rows1, s1).wait()
            pltpu.sync_copy(rows1,
                            out_hbm.at[pl.ds(base + (g + 1) * chunk, chunk)])
            return carry

        lax.fori_loop(0, nsteps // 2, pair, 0)

    return k(table, idx)


def _sc_presub(ma, mb, b2a3, b2revb3):
    mesh = plsc.VectorSubcoreMesh(**_MESH)

    @functools.partial(
        pl.kernel, mesh=mesh,
        out_type=jax.ShapeDtypeStruct((B_PAD, H), jnp.float32),
        scratch_types=[
            pltpu.VMEM((_CB,), jnp.int32),
            pltpu.VMEM((_CB,), jnp.int32),
            pltpu.VMEM((_CB,), jnp.int32),
            pltpu.VMEM((_CB,), jnp.int32),
            pltpu.VMEM((_CB, H), jnp.float32),
            pltpu.VMEM((_CB, H), jnp.float32),
            pltpu.VMEM((_CB, H), jnp.float32),
            pltpu.VMEM((_CB, H), jnp.float32),
            pltpu.VMEM((_CB, H), jnp.float32),
            pltpu.SemaphoreType.DMA,
            pltpu.SemaphoreType.DMA,
            pltpu.SemaphoreType.DMA,
            pltpu.SemaphoreType.DMA,
        ],
    )
    def k(ma_hbm, mb_hbm, ia_hbm, ib_hbm, out_hbm, ixa_v, ixb_v,
          ixa1_v, ixb1_v, ra0, rb0, ra1, rb1, out_v, sa0, sb0, sa1, sb1):
        wid = _wid()
        rbase = wid * (_NS * _CB)

        def compute(ra, rb):
            for i in range(_CB):
                for l in range(8):
                    sl = pl.ds(l * 16, 16)
                    out_v[i, sl] = ra[i, sl] - rb[i, sl]

        def start(g, ixa, ixb, ra, rb, sa, sb):
            off = rbase + g * _CB
            pltpu.sync_copy(ia_hbm.at[pl.ds(off, _CB)], ixa)
            pltpu.sync_copy(ib_hbm.at[pl.ds(off, _CB)], ixb)
            pltpu.async_copy(ma_hbm.at[ixa], ra, sa)
            pltpu.async_copy(mb_hbm.at[ixb], rb, sb)

        def finish(g, ixa, ixb, ra, rb, sa, sb):
            pltpu.make_async_copy(ma_hbm.at[ixa], ra, sa).wait()
            pltpu.make_async_copy(mb_hbm.at[ixb], rb, sb).wait()
            compute(ra, rb)
            pltpu.sync_copy(out_v, out_hbm.at[pl.ds(rbase + g * _CB, _CB)])

        start(0, ixa_v, ixb_v, ra0, rb0, sa0, sb0)

        def pair(i, carry):
            g = i * 2
            start(g + 1, ixa1_v, ixb1_v, ra1, rb1, sa1, sb1)
            finish(g, ixa_v, ixb_v, ra0, rb0, sa0, sb0)

            @pl.when(g + 2 < _NS)
            def _():
                start(g + 2, ixa_v, ixb_v, ra0, rb0, sa0, sb0)

            finish(g + 1, ixa1_v, ixb1_v, ra1, rb1, sa1, sb1)
            return carry

        lax.fori_loop(0, _NS // 2, pair, 0)

    return k(ma, mb, b2a3, b2revb3)


def _mm_relu_body(x_ref, wt_ref, o_ref):
    o_ref[...] = jax.nn.relu(
        lax.dot_general(x_ref[...], wt_ref[...],
                        (((0,), (0,)), ((), ())),
                        preferred_element_type=jnp.float32))


def _mm_relu(xT, wt, n_out, bn):
    k = xT.shape[0]
    return pl.pallas_call(
        _mm_relu_body,
        grid=(n_out // bn,),
        in_specs=[pl.BlockSpec((k, bn), lambda i: (0, i)),
                  pl.BlockSpec((k, H), lambda i: (0, 0))],
        out_specs=pl.BlockSpec((bn, H), lambda i: (i, 0)),
        out_shape=jax.ShapeDtypeStruct((n_out, H), jnp.float32),
    )(xT, wt)


_BA = 128


def _agg_base_body(nei_ref, base_ref, o_ref):
    x = nei_ref[...].reshape(_BA, 8, H)[:, :MAX_NB, :]
    o_ref[...] = base_ref[...] + x.sum(axis=1) * x.max(axis=1)


def _agg_nb_body(nei_ref, o_ref):
    x = nei_ref[...].reshape(_BA, 8, H)[:, :MAX_NB, :]
    o_ref[...] = x.sum(axis=1) * x.max(axis=1)


def _agg_base(nei8, base):
    return pl.pallas_call(
        _agg_base_body,
        grid=(A_PAD // _BA,),
        in_specs=[pl.BlockSpec((_BA * 8, H), lambda i: (i, 0)),
                  pl.BlockSpec((_BA, H), lambda i: (i, 0))],
        out_specs=pl.BlockSpec((_BA, H), lambda i: (i, 0)),
        out_shape=jax.ShapeDtypeStruct((A_PAD, H), jnp.float32),
    )(nei8, base)


def _agg_nb(nei8):
    return pl.pallas_call(
        _agg_nb_body,
        grid=(A_PAD // _BA,),
        in_specs=[pl.BlockSpec((_BA * 8, H), lambda i: (i, 0))],
        out_specs=pl.BlockSpec((_BA, H), lambda i: (i, 0)),
        out_shape=jax.ShapeDtypeStruct((A_PAD, H), jnp.float32),
    )(nei8)


def _bond_body(pre_ref, ib_ref, wt_ref, o_ref):
    o_ref[...] = jax.nn.relu(
        ib_ref[...] +
        jnp.dot(pre_ref[...], wt_ref[...], preferred_element_type=jnp.float32))


def _bond_update(pre, ib, wh_t):
    bn = 512
    return pl.pallas_call(
        _bond_body,
        grid=(B_PAD // bn,),
        in_specs=[pl.BlockSpec((bn, H), lambda i: (i, 0)),
                  pl.BlockSpec((bn, H), lambda i: (i, 0)),
                  pl.BlockSpec((H, H), lambda i: (0, 0))],
        out_specs=pl.BlockSpec((bn, H), lambda i: (i, 0)),
        out_shape=jax.ShapeDtypeStruct((B_PAD, H), jnp.float32),
    )(pre, ib, wh_t)


_BM5 = 8


def _node_body(agg_ref, ma_ref, ia_ref, l0_ref, l1_ref, l2_ref, gb_ref,
               wif_ref, bif_ref, wir_ref, bir_ref,
               gif_ref, gir_ref, h0_ref):
    node = (jnp.dot(agg_ref[...], l0_ref[...], preferred_element_type=jnp.float32)
            + jnp.dot(ma_ref[...], l1_ref[...], preferred_element_type=jnp.float32)
            + jnp.dot(ia_ref[...], l2_ref[...], preferred_element_type=jnp.float32))
    h0_ref[...] = node.reshape(_BM5, APM, H).max(axis=1)
    msg = jax.nn.relu(node + gb_ref[...])
    gif = jnp.dot(msg, wif_ref[...], preferred_element_type=jnp.float32) + bif_ref[...]
    gir = jnp.dot(msg, wir_ref[...], preferred_element_type=jnp.float32) + bir_ref[...]
    gif_ref[...] = gif.reshape(_BM5, APM, 3 * H).swapaxes(0, 1)
    gir_ref[...] = gir.reshape(_BM5, APM, 3 * H).swapaxes(0, 1)


def _node_project(agg, ma, ia, l0, l1, l2, gbias, wif, bif, wir, bir):
    rows = _BM5 * APM
    wspec = pl.BlockSpec((H, H), lambda i: (0, 0))
    w3spec = pl.BlockSpec((H, 3 * H), lambda i: (0, 0))
    b3spec = pl.BlockSpec((1, 3 * H), lambda i: (0, 0))
    return pl.pallas_call(
        _node_body,
        grid=(N_MOL // _BM5,),
        in_specs=[pl.BlockSpec((rows, H), lambda i: (i, 0)),
                  pl.BlockSpec((rows, H), lambda i: (i, 0)),
                  pl.BlockSpec((rows, H), lambda i: (i, 0)),
                  wspec, wspec, wspec,
                  pl.BlockSpec((1, H), lambda i: (0, 0)),
                  w3spec, b3spec, w3spec, b3spec],
        out_specs=[pl.BlockSpec((APM, _BM5, 3 * H), lambda i: (0, i, 0)),
                   pl.BlockSpec((APM, _BM5, 3 * H), lambda i: (0, i, 0)),
                   pl.BlockSpec((_BM5, H), lambda i: (i, 0))],
        out_shape=[jax.ShapeDtypeStruct((APM, N_MOL, 3 * H), jnp.float32),
                   jax.ShapeDtypeStruct((APM, N_MOL, 3 * H), jnp.float32),
                   jax.ShapeDtypeStruct((N_MOL, H), jnp.float32)],
    )(agg, ma, ia, l0, l1, l2, gbias, wif, bif, wir, bir)


def _gru_body(gif_ref, gir_ref, h0_ref, whf_ref, bhf_ref, whr_ref, bhr_ref,
              of_ref, or_ref, hf_s, hr_s):
    t = pl.program_id(0)

    @pl.when(t == 0)
    def _():
        hf_s[...] = h0_ref[...]
        hr_s[...] = h0_ref[...]

    def step(gi, h, wh_ref, bh_ref):
        gh = jnp.dot(h, wh_ref[...], preferred_element_type=jnp.float32) + bh_ref[...]
        r = jax.nn.sigmoid(gi[:, :H] + gh[:, :H])
        z = jax.nn.sigmoid(gi[:, H:2 * H] + gh[:, H:2 * H])
        n = jnp.tanh(gi[:, 2 * H:] + r * gh[:, 2 * H:])
        return (1.0 - z) * n + z * h

    hf = step(gif_ref[...].reshape(N_MOL, 3 * H), hf_s[...], whf_ref, bhf_ref)
    hr = step(gir_ref[...].reshape(N_MOL, 3 * H), hr_s[...], whr_ref, bhr_ref)
    hf_s[...] = hf
    hr_s[...] = hr
    of_ref[...] = hf.reshape(1, N_MOL, H)
    or_ref[...] = hr.reshape(1, N_MOL, H)


def _gru(gif, gir, h0, whf, bhf, whr, bhr):
    w3spec = pl.BlockSpec((H, 3 * H), lambda t: (0, 0))
    b3spec = pl.BlockSpec((1, 3 * H), lambda t: (0, 0))
    return pl.pallas_call(
        _gru_body,
        grid=(APM,),
        in_specs=[pl.BlockSpec((1, N_MOL, 3 * H), lambda t: (t, 0, 0)),
                  pl.BlockSpec((1, N_MOL, 3 * H), lambda t: (APM - 1 - t, 0, 0)),
                  pl.BlockSpec((N_MOL, H), lambda t: (0, 0)),
                  w3spec, b3spec, w3spec, b3spec],
        out_specs=[pl.BlockSpec((1, N_MOL, H), lambda t: (t, 0, 0)),
                   pl.BlockSpec((1, N_MOL, H), lambda t: (APM - 1 - t, 0, 0))],
        out_shape=[jax.ShapeDtypeStruct((APM, N_MOL, H), jnp.float32),
                   jax.ShapeDtypeStruct((APM, N_MOL, H), jnp.float32)],
        scratch_shapes=[pltpu.VMEM((N_MOL, H), jnp.float32),
                        pltpu.VMEM((N_MOL, H), jnp.float32)],
    )(gif, gir, h0, whf, bhf, whr, bhr)


_BM7 = 128


def _readout_body(of_ref, or_ref, wof_ref, wor_ref, b_ref, o_ref):
    f = of_ref[...].reshape(APM * _BM7, H)
    r = or_ref[...].reshape(APM * _BM7, H)
    ah = jax.nn.relu(
        jnp.dot(f, wof_ref[...], preferred_element_type=jnp.float32)
        + jnp.dot(r, wor_ref[...], preferred_element_type=jnp.float32)
        + b_ref[...])
    o_ref[...] = ah.reshape(APM, _BM7, H).sum(axis=0) * (1.0 / APM)


def _readout(of, orr, wof, wor, b):
    wspec = pl.BlockSpec((H, H), lambda i: (0, 0))
    return pl.pallas_call(
        _readout_body,
        grid=(N_MOL // _BM7,),
        in_specs=[pl.BlockSpec((APM, _BM7, H), lambda i: (0, i, 0)),
                  pl.BlockSpec((APM, _BM7, H), lambda i: (0, i, 0)),
                  wspec, wspec,
                  pl.BlockSpec((1, H), lambda i: (0, 0))],
        out_specs=pl.BlockSpec((_BM7, H), lambda i: (i, 0)),
        out_shape=jax.ShapeDtypeStruct((N_MOL, H), jnp.float32),
    )(of, orr, wof, wor, b)


def kernel(f_atoms, f_bonds, a2b, b2a, b2revb, a_scope, W_i_atom, W_i_bond,
           W_h_0, W_h_1, lr_W, W_o_W, W_o_b, gru_bias, W_ih_f, W_hh_f,
           b_ih_f, b_hh_f, W_ih_r, W_hh_r, b_ih_r, b_hh_r):
    del a_scope

    faT = jnp.pad(f_atoms.T, ((0, 3), (0, A_PAD - N_ATOMS)))
    fbT = jnp.pad(f_bonds.T, ((0, 5), (0, B_PAD - N_BONDS)))
    wia_t = jnp.pad(W_i_atom.T, ((0, 3), (0, 0)))
    wib_t = jnp.pad(W_i_bond.T, ((0, 5), (0, 0)))
    ia = _mm_relu(faT, wia_t, A_PAD, 512)
    ib = _mm_relu(fbT, wib_t, B_PAD, 512)

    arow = jnp.arange(A_PAD, dtype=jnp.int32)[:, None]
    a2b_p = jnp.pad(a2b.astype(jnp.int32), ((0, A_PAD - N_ATOMS), (0, 0)))
    a2b_p = jnp.where(arow < N_ATOMS, a2b_p, arow)
    a2b8 = jnp.concatenate(
        [a2b_p, jnp.broadcast_to(arow, (A_PAD, 2))], axis=1).reshape(-1)
    brow = jnp.arange(B_PAD, dtype=jnp.int32)
    b2a3 = jnp.where(brow < N_BONDS,
                     jnp.pad(b2a.astype(jnp.int32), (0, B_PAD - N_BONDS)),
                     brow % jnp.int32(N_ATOMS))
    b2revb3 = jnp.where(brow < N_BONDS,
                        jnp.pad(b2revb.astype(jnp.int32), (0, B_PAD - N_BONDS)),
                        brow % jnp.int32(N_BONDS))

    ma = ia
    mb = ib
    for wh in (W_h_0, W_h_1):
        ma = _agg_base(_sc_gather(mb, a2b8, 224), ma)
        pre = _sc_presub(ma, mb, b2a3, b2revb3)
        mb = _bond_update(pre, ib, wh.T)

    agg = _agg_nb(_sc_gather(mb, a2b8, 224))

    rows = N_MOL * APM
    agg_s = lax.dynamic_slice(agg, (1, 0), (rows, H))
    ma_s = lax.dynamic_slice(ma, (1, 0), (rows, H))
    ia_s = lax.dynamic_slice(ia, (1, 0), (rows, H))

    l0 = lr_W[:, :H].T
    l1 = lr_W[:, H:2 * H].T
    l2 = lr_W[:, 2 * H:].T
    gif, gir, h0 = _node_project(
        agg_s, ma_s, ia_s, l0, l1, l2, gru_bias.reshape(1, H),
        W_ih_f.T, b_ih_f.reshape(1, 3 * H), W_ih_r.T, b_ih_r.reshape(1, 3 * H))

    of, orr = _gru(gif, gir, h0, W_hh_f.T, b_hh_f.reshape(1, 3 * H),
                   W_hh_r.T, b_hh_r.reshape(1, 3 * H))

    return _readout(of, orr, W_o_W[:, :H].T, W_o_W[:, H:].T,
                    W_o_b.reshape(1, H))
